# Initial kernel scaffold; baseline (speedup 1.0000x reference)
import numpy as np
import jax
import jax.numpy as jnp
from jax import lax
from jax.experimental import pallas as pl

_N = 8192
_K = 32
_G = 50
_CUT2 = 100.0
_BIG = 1e10
_CENTERS_NP = np.linspace(0.0, 10.0, _G, dtype=np.float32)
_SPACING = -0.5 / float(_CENTERS_NP[1] - _CENTERS_NP[0]) ** 2

_R = 256
_B2 = 2048


def _select_body(xt_ref, pos_ref, src_ref, dist_ref, valid_ref):
    i = pl.program_id(0)
    r0 = i * _R
    xr = xt_ref[0:1, :]
    yr = xt_ref[1:2, :]
    zr = xt_ref[2:3, :]
    sqr = xr * xr + yr * yr + zr * zr
    pr = pos_ref[...]
    xi = pr[:, 0:1]
    yi = pr[:, 1:2]
    zi = pr[:, 2:3]
    sqi = xi * xi + yi * yi + zi * zi
    dot = xi * xr + yi * yr + zi * zr
    d2 = sqi + sqr - 2.0 * dot
    d2 = jnp.maximum(d2, 0.0)
    col = lax.broadcasted_iota(jnp.int32, (_R, _N), 1)
    rowi = r0 + lax.broadcasted_iota(jnp.int32, (_R, _N), 0)
    d2 = jnp.where(col == rowi, _BIG, d2)
    d2 = jnp.where(d2 > _CUT2, _BIG, d2)

    vals = []
    idxs = []
    for _ in range(_K):
        m = jnp.min(d2, axis=1, keepdims=True)
        am = jnp.min(jnp.where(d2 == m, col, _N), axis=1,
                     keepdims=True)
        vals.append(m)
        idxs.append(am)
        d2 = jnp.where(col == am, _BIG, d2)
    v = jnp.concatenate(vals, axis=1)
    ix = jnp.concatenate(idxs, axis=1)
    valid = v < _BIG * 0.5
    rows32 = r0 + lax.broadcasted_iota(jnp.int32, (_R, _K), 0)
    src_ref[...] = jnp.where(valid, ix, rows32)
    dist_ref[...] = jnp.where(valid, jnp.sqrt(jnp.maximum(v, 0.0)), 0.0)
    valid_ref[...] = valid.astype(jnp.float32)


def _rbf_body(d_ref, v_ref, c_ref, out_ref):
    cd = d_ref[...] - c_ref[...]
    out_ref[...] = jnp.exp(_SPACING * (cd * cd)) * v_ref[...]


def kernel(positions):
    xt = positions.T
    src, dist, validf = pl.pallas_call(
        _select_body,
        grid=(_N // _R,),
        in_specs=[
            pl.BlockSpec((3, _N), lambda i: (0, 0)),
            pl.BlockSpec((_R, 3), lambda i: (i, 0)),
        ],
        out_specs=[
            pl.BlockSpec((_R, _K), lambda i: (i, 0)),
            pl.BlockSpec((_R, _K), lambda i: (i, 0)),
            pl.BlockSpec((_R, _K), lambda i: (i, 0)),
        ],
        out_shape=[
            jax.ShapeDtypeStruct((_N, _K), jnp.int32),
            jax.ShapeDtypeStruct((_N, _K), jnp.float32),
            jax.ShapeDtypeStruct((_N, _K), jnp.float32),
        ],
    )(xt, positions)

    e = _N * _K
    dcol = dist.reshape(e, 1)
    vcol = validf.reshape(e, 1)
    centers = jnp.asarray(_CENTERS_NP).reshape(1, _G)
    rbf = pl.pallas_call(
        _rbf_body,
        grid=(e // _B2,),
        in_specs=[
            pl.BlockSpec((_B2, 1), lambda i: (i, 0)),
            pl.BlockSpec((_B2, 1), lambda i: (i, 0)),
            pl.BlockSpec((1, _G), lambda i: (0, 0)),
        ],
        out_specs=pl.BlockSpec((_B2, _G), lambda i: (i, 0)),
        out_shape=jax.ShapeDtypeStruct((e, _G), jnp.float32),
    )(dcol, vcol, centers)

    dst = jnp.repeat(jnp.arange(_N, dtype=jnp.int32), _K)
    edges = jnp.stack([src.reshape(-1), dst], axis=0)
    return edges, dist.reshape(-1), rbf


# speedup vs baseline: 3.7932x; 3.7932x over previous
import numpy as np
import jax
import jax.numpy as jnp
from jax import lax
from jax.experimental import pallas as pl

_N = 8192
_K = 32
_G = 50
_CUT2 = 100.0
_BIG = 1e10
_CENTERS_NP = np.linspace(0.0, 10.0, _G, dtype=np.float32)
_SPACING = -0.5 / float(_CENTERS_NP[1] - _CENTERS_NP[0]) ** 2

_R = 256
_B2 = 2048


def _select_body(xt_ref, pos_ref, sqr_ref, sqc_ref, src_ref, dist_ref,
                 valid_ref):
    i = pl.program_id(0)
    r0 = i * _R
    xr = xt_ref[0:1, :]
    yr = xt_ref[1:2, :]
    zr = xt_ref[2:3, :]
    pr = pos_ref[...]
    xi = pr[:, 0:1]
    yi = pr[:, 1:2]
    zi = pr[:, 2:3]
    sqi = sqr_ref[...]
    sqr = sqc_ref[...]

    def b16(t):
        return t.astype(jnp.bfloat16).astype(jnp.float32)

    xx = b16(xi) * b16(xr)
    yy = b16(yi) * b16(yr)
    zz = b16(zi) * b16(zr)
    s1 = xx + yy
    ap = s1 - yy
    e1 = (xx - ap) + (yy - (s1 - ap))
    s2 = s1 + zz
    ap2 = s2 - zz
    e2 = (s1 - ap2) + (zz - (s2 - ap2))
    dotb = s2 + (e1 + e2)
    d2 = (sqi + sqr) - 2.0 * dotb
    d2 = jnp.maximum(d2, 0.0)
    col = lax.broadcasted_iota(jnp.int32, (_R, _N), 1)
    rowi = r0 + lax.broadcasted_iota(jnp.int32, (_R, _N), 0)
    d2 = jnp.where(col == rowi, _BIG, d2)
    d2 = jnp.where(d2 > _CUT2, _BIG, d2)

    ax = xi - xr
    ay = yi - yr
    az = zi - zr
    d2a = (ax * ax + ay * ay) + az * az

    vals = []
    idxs = []
    accs = []
    for _ in range(_K):
        m = jnp.min(d2, axis=1, keepdims=True)
        hit = d2 == m
        am = jnp.min(jnp.where(hit, col, _N), axis=1,
                     keepdims=True)
        sel = col == am
        va = jnp.min(jnp.where(sel, d2a, _BIG), axis=1,
                     keepdims=True)
        vals.append(m)
        idxs.append(am)
        accs.append(va)
        d2 = jnp.where(sel, _BIG, d2)
    v = jnp.concatenate(vals, axis=1)
    ix = jnp.concatenate(idxs, axis=1)
    va = jnp.concatenate(accs, axis=1)
    valid = v < _BIG * 0.5
    rows32 = r0 + lax.broadcasted_iota(jnp.int32, (_R, _K), 0)
    src_ref[...] = jnp.where(valid, ix, rows32)
    dist_ref[...] = jnp.where(valid & (va > 0.0), jnp.sqrt(va), 0.0)
    valid_ref[...] = valid.astype(jnp.float32)


def _rbf_body(d_ref, v_ref, c_ref, out_ref):
    cd = d_ref[...] - c_ref[...]
    out_ref[...] = jnp.exp(_SPACING * (cd * cd)) * v_ref[...]


def kernel(positions):
    xt = positions.T
    sq = jnp.sum(positions * positions, axis=1)
    src, dist, validf = pl.pallas_call(
        _select_body,
        grid=(_N // _R,),
        in_specs=[
            pl.BlockSpec((3, _N), lambda i: (0, 0)),
            pl.BlockSpec((_R, 3), lambda i: (i, 0)),
            pl.BlockSpec((_R, 1), lambda i: (i, 0)),
            pl.BlockSpec((1, _N), lambda i: (0, 0)),
        ],
        out_specs=[
            pl.BlockSpec((_R, _K), lambda i: (i, 0)),
            pl.BlockSpec((_R, _K), lambda i: (i, 0)),
            pl.BlockSpec((_R, _K), lambda i: (i, 0)),
        ],
        out_shape=[
            jax.ShapeDtypeStruct((_N, _K), jnp.int32),
            jax.ShapeDtypeStruct((_N, _K), jnp.float32),
            jax.ShapeDtypeStruct((_N, _K), jnp.float32),
        ],
    )(xt, positions, sq.reshape(_N, 1), sq.reshape(1, _N))

    e = _N * _K
    dcol = dist.reshape(e, 1)
    vcol = validf.reshape(e, 1)
    centers = jnp.asarray(_CENTERS_NP).reshape(1, _G)
    rbf = pl.pallas_call(
        _rbf_body,
        grid=(e // _B2,),
        in_specs=[
            pl.BlockSpec((_B2, 1), lambda i: (i, 0)),
            pl.BlockSpec((_B2, 1), lambda i: (i, 0)),
            pl.BlockSpec((1, _G), lambda i: (0, 0)),
        ],
        out_specs=pl.BlockSpec((_B2, _G), lambda i: (i, 0)),
        out_shape=jax.ShapeDtypeStruct((e, _G), jnp.float32),
    )(dcol, vcol, centers)

    dst = jnp.repeat(jnp.arange(_N, dtype=jnp.int32), _K)
    edges = jnp.stack([src.reshape(-1), dst], axis=0)
    return edges, dist.reshape(-1), rbf
